# baseline (speedup 1.0000x reference)
import functools

import jax
import jax.numpy as jnp
from jax.experimental import pallas as pl
from jax.experimental.pallas import tpu as pltpu


def _fused_kernel(x_ref, mu_ref, ls_ref, eps_ref, mub_ref, lsb_ref, epsb_ref,
                  o_ref, w_ref, *, n_kc):
    j = pl.program_id(1)
    kc = pl.program_id(2)
    tk = x_ref.shape[1]

    @pl.when(j == 0)
    def _():
        w = mu_ref[...] + jnp.exp(ls_ref[...]) * eps_ref[...]
        w_ref[pl.ds(kc * tk, tk), :] = w.astype(jnp.bfloat16).T

    xv = x_ref[...].astype(jnp.bfloat16)
    part = jnp.dot(xv, w_ref[pl.ds(kc * tk, tk), :],
                   preferred_element_type=jnp.float32)

    @pl.when(kc == 0)
    def _():
        bias = mub_ref[...] + jnp.exp(lsb_ref[...]) * epsb_ref[...]
        o_ref[...] = part + bias

    @pl.when(kc > 0)
    def _():
        o_ref[...] += part

    del n_kc


def kernel(x, mu_w, log_sigma_w, eps_w, mu_b, log_sigma_b, eps_b):
    OUT, IN = mu_w.shape
    orig_shape = x.shape
    x2 = x.reshape(-1, IN)
    B = x2.shape[0]

    tb = min(512, B)
    tk = min(512, IN)
    n_kc = IN // tk
    n_b = B // tb
    n_c = 2 if n_b % 2 == 0 else 1
    jp = n_b // n_c

    last = n_kc - 1

    def param_idx(c, j, kc):
        return (0, jnp.where(j == 0, kc, last))

    body = functools.partial(_fused_kernel, n_kc=n_kc)

    y = pl.pallas_call(
        body,
        out_shape=jax.ShapeDtypeStruct((B, OUT), x.dtype),
        grid=(n_c, jp, n_kc),
        in_specs=[
            pl.BlockSpec((tb, tk), lambda c, j, kc: (c * jp + j, kc)),
            pl.BlockSpec((OUT, tk), param_idx),
            pl.BlockSpec((OUT, tk), param_idx),
            pl.BlockSpec((OUT, tk), param_idx),
            pl.BlockSpec((1, OUT), lambda c, j, kc: (0, 0)),
            pl.BlockSpec((1, OUT), lambda c, j, kc: (0, 0)),
            pl.BlockSpec((1, OUT), lambda c, j, kc: (0, 0)),
        ],
        out_specs=pl.BlockSpec((tb, OUT), lambda c, j, kc: (c * jp + j, 0)),
        scratch_shapes=[pltpu.VMEM((IN, OUT), jnp.bfloat16)],
        compiler_params=pltpu.CompilerParams(
            dimension_semantics=("parallel", "arbitrary", "arbitrary"),
            vmem_limit_bytes=60 * 1024 * 1024),
        cost_estimate=pl.CostEstimate(
            flops=2 * B * IN * OUT,
            transcendentals=2 * IN * OUT,
            bytes_accessed=4 * (B * IN + B * OUT) + 4 * 6 * IN * OUT),
    )(x2, mu_w, log_sigma_w, eps_w,
      mu_b.reshape(1, OUT), log_sigma_b.reshape(1, OUT),
      eps_b.reshape(1, OUT))

    return y.reshape(*orig_shape[:-1], OUT)


# speedup vs baseline: 1.3495x; 1.3495x over previous
import functools

import jax
import jax.numpy as jnp
from jax.experimental import pallas as pl
from jax.experimental.pallas import tpu as pltpu


def _fused_kernel(x_ref, mu_ref, ls_ref, eps_ref, mub_ref, lsb_ref, epsb_ref,
                  o_ref, w_ref, *, n_kc):
    s = pl.program_id(1)
    tk = mu_ref.shape[1]

    @pl.when(s < n_kc)
    def _():
        sk = jnp.minimum(s, n_kc - 1)
        w = mu_ref[...] + jnp.exp(ls_ref[...]) * eps_ref[...]
        w_ref[pl.ds(sk * tk, tk), :] = w.astype(jnp.bfloat16).T

    @pl.when(s >= n_kc)
    def _():
        xv = x_ref[...].astype(jnp.bfloat16)
        acc = jnp.dot(xv, w_ref[...], preferred_element_type=jnp.float32)
        bias = mub_ref[...] + jnp.exp(lsb_ref[...]) * epsb_ref[...]
        o_ref[...] = acc + bias


def kernel(x, mu_w, log_sigma_w, eps_w, mu_b, log_sigma_b, eps_b):
    OUT, IN = mu_w.shape
    orig_shape = x.shape
    x2 = x.reshape(-1, IN)
    B = x2.shape[0]

    tb = min(512, B)
    tk = min(256, IN)
    n_kc = IN // tk
    n_b = B // tb
    n_c = 2 if n_b % 2 == 0 else 1
    jp = n_b // n_c

    def param_idx(c, s):
        return (0, jnp.minimum(s, n_kc - 1))

    def x_idx(c, s):
        return (c * jp + jnp.maximum(s - n_kc, 0), 0)

    body = functools.partial(_fused_kernel, n_kc=n_kc)

    y = pl.pallas_call(
        body,
        out_shape=jax.ShapeDtypeStruct((B, OUT), x.dtype),
        grid=(n_c, n_kc + jp),
        in_specs=[
            pl.BlockSpec((tb, IN), x_idx),
            pl.BlockSpec((OUT, tk), param_idx),
            pl.BlockSpec((OUT, tk), param_idx),
            pl.BlockSpec((OUT, tk), param_idx),
            pl.BlockSpec((1, OUT), lambda c, s: (0, 0)),
            pl.BlockSpec((1, OUT), lambda c, s: (0, 0)),
            pl.BlockSpec((1, OUT), lambda c, s: (0, 0)),
        ],
        out_specs=pl.BlockSpec((tb, OUT), x_idx),
        scratch_shapes=[pltpu.VMEM((IN, OUT), jnp.bfloat16)],
        compiler_params=pltpu.CompilerParams(
            dimension_semantics=("parallel", "arbitrary"),
            vmem_limit_bytes=60 * 1024 * 1024),
        cost_estimate=pl.CostEstimate(
            flops=2 * B * IN * OUT,
            transcendentals=2 * IN * OUT,
            bytes_accessed=4 * (B * IN + B * OUT) + 4 * 6 * IN * OUT),
    )(x2, mu_w, log_sigma_w, eps_w,
      mu_b.reshape(1, OUT), log_sigma_b.reshape(1, OUT),
      eps_b.reshape(1, OUT))

    return y.reshape(*orig_shape[:-1], OUT)


# speedup vs baseline: 1.4877x; 1.1025x over previous
import jax
import jax.numpy as jnp
from jax.experimental import pallas as pl
from jax.experimental.pallas import tpu as pltpu


def _reparam_t_kernel(mu_ref, ls_ref, eps_ref, wt_ref):
    w = mu_ref[...] + jnp.exp(ls_ref[...]) * eps_ref[...]
    wt_ref[...] = w.astype(jnp.bfloat16).T


def _matmul_bias_kernel(x_ref, wt_ref, mub_ref, lsb_ref, epsb_ref, o_ref):
    xv = x_ref[...].astype(jnp.bfloat16)
    acc = jnp.dot(xv, wt_ref[...], preferred_element_type=jnp.float32)
    bias = mub_ref[...] + jnp.exp(lsb_ref[...]) * epsb_ref[...]
    o_ref[...] = acc + bias


def kernel(x, mu_w, log_sigma_w, eps_w, mu_b, log_sigma_b, eps_b):
    OUT, IN = mu_w.shape
    orig_shape = x.shape
    x2 = x.reshape(-1, IN)
    B = x2.shape[0]

    tn = min(512, OUT)
    tk = min(2048, IN)
    tb = min(1024, B)

    w_t = pl.pallas_call(
        _reparam_t_kernel,
        out_shape=jax.ShapeDtypeStruct((IN, OUT), jnp.bfloat16),
        grid=(OUT // tn, IN // tk),
        in_specs=[
            pl.BlockSpec((tn, tk), lambda n, k: (n, k)),
            pl.BlockSpec((tn, tk), lambda n, k: (n, k)),
            pl.BlockSpec((tn, tk), lambda n, k: (n, k)),
        ],
        out_specs=pl.BlockSpec((tk, tn), lambda n, k: (k, n)),
        compiler_params=pltpu.CompilerParams(
            dimension_semantics=("parallel", "arbitrary")),
    )(mu_w, log_sigma_w, eps_w)

    y = pl.pallas_call(
        _matmul_bias_kernel,
        out_shape=jax.ShapeDtypeStruct((B, OUT), x.dtype),
        grid=(B // tb,),
        in_specs=[
            pl.BlockSpec((tb, IN), lambda i: (i, 0)),
            pl.BlockSpec((IN, OUT), lambda i: (0, 0)),
            pl.BlockSpec((1, OUT), lambda i: (0, 0)),
            pl.BlockSpec((1, OUT), lambda i: (0, 0)),
            pl.BlockSpec((1, OUT), lambda i: (0, 0)),
        ],
        out_specs=pl.BlockSpec((tb, OUT), lambda i: (i, 0)),
        compiler_params=pltpu.CompilerParams(
            dimension_semantics=("parallel",),
            vmem_limit_bytes=60 * 1024 * 1024),
    )(x2, w_t, mu_b.reshape(1, OUT), log_sigma_b.reshape(1, OUT),
      eps_b.reshape(1, OUT))

    return y.reshape(*orig_shape[:-1], OUT)


# speedup vs baseline: 1.4880x; 1.0002x over previous
import jax
import jax.numpy as jnp
from jax.experimental import pallas as pl
from jax.experimental.pallas import tpu as pltpu


def _reparam_t_kernel(mu_ref, ls_ref, eps_ref, wt_ref):
    w = mu_ref[...] + jnp.exp(ls_ref[...]) * eps_ref[...]
    wt_ref[...] = w.astype(jnp.bfloat16).T


def _matmul_bias_kernel(x_ref, wt_ref, mub_ref, lsb_ref, epsb_ref, o_ref):
    xv = x_ref[...].astype(jnp.bfloat16)
    acc = jnp.dot(xv, wt_ref[...], preferred_element_type=jnp.float32)
    bias = mub_ref[...] + jnp.exp(lsb_ref[...]) * epsb_ref[...]
    o_ref[...] = acc + bias


def kernel(x, mu_w, log_sigma_w, eps_w, mu_b, log_sigma_b, eps_b):
    OUT, IN = mu_w.shape
    orig_shape = x.shape
    x2 = x.reshape(-1, IN)
    B = x2.shape[0]

    tn = min(256, OUT)
    tk = min(2048, IN)
    tb = min(1024, B)

    w_t = pl.pallas_call(
        _reparam_t_kernel,
        out_shape=jax.ShapeDtypeStruct((IN, OUT), jnp.bfloat16),
        grid=(OUT // tn, IN // tk),
        in_specs=[
            pl.BlockSpec((tn, tk), lambda n, k: (n, k)),
            pl.BlockSpec((tn, tk), lambda n, k: (n, k)),
            pl.BlockSpec((tn, tk), lambda n, k: (n, k)),
        ],
        out_specs=pl.BlockSpec((tk, tn), lambda n, k: (k, n)),
        compiler_params=pltpu.CompilerParams(
            dimension_semantics=("parallel", "arbitrary")),
    )(mu_w, log_sigma_w, eps_w)

    y = pl.pallas_call(
        _matmul_bias_kernel,
        out_shape=jax.ShapeDtypeStruct((B, OUT), x.dtype),
        grid=(B // tb,),
        in_specs=[
            pl.BlockSpec((tb, IN), lambda i: (i, 0)),
            pl.BlockSpec((IN, OUT), lambda i: (0, 0)),
            pl.BlockSpec((1, OUT), lambda i: (0, 0)),
            pl.BlockSpec((1, OUT), lambda i: (0, 0)),
            pl.BlockSpec((1, OUT), lambda i: (0, 0)),
        ],
        out_specs=pl.BlockSpec((tb, OUT), lambda i: (i, 0)),
        compiler_params=pltpu.CompilerParams(
            dimension_semantics=("parallel",),
            vmem_limit_bytes=60 * 1024 * 1024),
    )(x2, w_t, mu_b.reshape(1, OUT), log_sigma_b.reshape(1, OUT),
      eps_b.reshape(1, OUT))

    return y.reshape(*orig_shape[:-1], OUT)
